# Initial kernel scaffold; baseline (speedup 1.0000x reference)
import jax
import jax.numpy as jnp
from jax.experimental import pallas as pl


def kernel(user_indices, item_indices, edge_index, weights, user_emb, item_emb):
    raise NotImplementedError("write your pallas kernel here")


# speedup vs baseline: 2.7899x; 2.7899x over previous
import functools

import jax
import jax.numpy as jnp
from jax import lax
from jax.experimental import pallas as pl
from jax.experimental.pallas import tpu as pltpu
from jax.experimental.pallas import tpu_sc as plsc

N_NODES = 10000
N_PAD = 10240
D = 128
E = 320000
B = 4096

NC = 2
NS = 16
L = 16

E_PER_TILE = E // NS
CHUNK = 80
N_CHUNKS = E_PER_TILE // CHUNK
ROWS_PER_TILE = N_PAD // NS
WB = 128

_mesh = plsc.VectorSubcoreMesh(core_axis_name="c", subcore_axis_name="s")


def _scale_rows(rows_v, wv_v, n_rows):
    def body(r, _):
        widx = jnp.full((L,), r, dtype=jnp.int32)
        wb = plsc.load_gather(wv_v, [widx])
        for j in range(D // L):
            sl = pl.ds(j * L, L)
            rows_v[r, sl] = rows_v[r, sl] * wb
        return 0

    lax.fori_loop(0, n_rows, body, 0, unroll=2)


def _layer_body(u_emb, i_emb, src, dst, w_hbm, u_out, i_out,
                acc, gidx_v, sidx_v, wv_v, rows_v, zb, sem):
    c = lax.axis_index("c")
    s = lax.axis_index("s")

    def zrow(r, _):
        for j in range(D // L):
            zb[r, pl.ds(j * L, L)] = jnp.zeros((L,), jnp.float32)
        return 0

    lax.fori_loop(0, WB, zrow, 0)
    for k in range(ROWS_PER_TILE // WB):
        pltpu.sync_copy(zb, acc.at[pl.ds(s * ROWS_PER_TILE + k * WB, WB)])
    plsc.subcore_barrier()

    def do_pass(gather_tab, scat_idx, gath_idx, out_hbm):
        base0 = s * E_PER_TILE

        def step(it, _):
            base = base0 + it * CHUNK
            pltpu.sync_copy(gath_idx.at[pl.ds(base, CHUNK)], gidx_v)
            pltpu.sync_copy(scat_idx.at[pl.ds(base, CHUNK)], sidx_v)
            pltpu.sync_copy(w_hbm.at[pl.ds(base, CHUNK)], wv_v)
            pltpu.async_copy(gather_tab.at[gidx_v], rows_v, sem).wait()
            _scale_rows(rows_v, wv_v, CHUNK)
            pltpu.sync_copy(rows_v, acc.at[sidx_v], add=True)
            return 0

        lax.fori_loop(0, N_CHUNKS, step, 0)
        plsc.subcore_barrier()
        for k in range(ROWS_PER_TILE // WB):
            row0 = s * ROWS_PER_TILE + k * WB
            pltpu.sync_copy(acc.at[pl.ds(row0, WB)], zb)
            pltpu.sync_copy(zb, out_hbm.at[pl.ds(row0, WB)])

    @pl.when(c == 0)
    def _():
        do_pass(i_emb, src, dst, u_out)

    @pl.when(c == 1)
    def _():
        do_pass(u_emb, dst, src, i_out)


@jax.jit
def _layer(u_emb, i_emb, src, dst, weights):
    f = pl.kernel(
        functools.partial(_layer_body),
        compiler_params=pltpu.CompilerParams(needs_layout_passes=False),
        out_type=(
            jax.ShapeDtypeStruct((N_PAD, D), jnp.float32),
            jax.ShapeDtypeStruct((N_PAD, D), jnp.float32),
        ),
        mesh=_mesh,
        scratch_types=[
            pltpu.VMEM_SHARED((N_PAD, D), jnp.float32),
            pltpu.VMEM((CHUNK,), jnp.int32),
            pltpu.VMEM((CHUNK,), jnp.int32),
            pltpu.VMEM((CHUNK,), jnp.float32),
            pltpu.VMEM((CHUNK, D), jnp.float32),
            pltpu.VMEM((WB, D), jnp.float32),
            pltpu.SemaphoreType.DMA,
        ],
    )
    return f(u_emb, i_emb, src, dst, weights)


P_PER_TILE = B // (NC * NS)
PC = 64


def _final_body(uidx_hbm, iidx_hbm, u0, u1, u2, u3, i0, i1, i2, i3, out_hbm,
                uidx_v, iidx_v, ubuf, ibuf, out_v, sem):
    c = lax.axis_index("c")
    s = lax.axis_index("s")
    wid = s * NC + c
    base = wid * P_PER_TILE

    utabs = (u0, u1, u2, u3)
    itabs = (i0, i1, i2, i3)
    for co in range(P_PER_TILE // PC):
        pltpu.sync_copy(uidx_hbm.at[pl.ds(base + co * PC, PC)], uidx_v)
        pltpu.sync_copy(iidx_hbm.at[pl.ds(base + co * PC, PC)], iidx_v)
        for t in range(4):
            pltpu.async_copy(utabs[t].at[uidx_v], ubuf.at[t], sem).wait()
            pltpu.async_copy(itabs[t].at[iidx_v], ibuf.at[t], sem).wait()
        for g in range(PC // L):
            rows = lax.iota(jnp.int32, L) + g * L

            def dstep(d, acc):
                col = jnp.full((L,), d, dtype=jnp.int32)
                us = plsc.load_gather(ubuf, [jnp.zeros((L,), jnp.int32), rows, col])
                is_ = plsc.load_gather(ibuf, [jnp.zeros((L,), jnp.int32), rows, col])
                for t in range(1, 4):
                    tt = jnp.full((L,), t, dtype=jnp.int32)
                    us = us + plsc.load_gather(ubuf, [tt, rows, col])
                    is_ = is_ + plsc.load_gather(ibuf, [tt, rows, col])
                return acc + us * is_

            acc = lax.fori_loop(0, D, dstep, jnp.zeros((L,), jnp.float32),
                                unroll=2)
            out_v[pl.ds(co * PC + g * L, L)] = acc * (1.0 / 16.0)
    pltpu.sync_copy(out_v, out_hbm.at[pl.ds(base, P_PER_TILE)])


@jax.jit
def _finalize(user_indices, item_indices, utabs, itabs):
    f = pl.kernel(
        _final_body,
        compiler_params=pltpu.CompilerParams(needs_layout_passes=False),
        out_type=jax.ShapeDtypeStruct((B,), jnp.float32),
        mesh=_mesh,
        scratch_types=[
            pltpu.VMEM((PC,), jnp.int32),
            pltpu.VMEM((PC,), jnp.int32),
            pltpu.VMEM((4, PC, D), jnp.float32),
            pltpu.VMEM((4, PC, D), jnp.float32),
            pltpu.VMEM((P_PER_TILE,), jnp.float32),
            pltpu.SemaphoreType.DMA,
        ],
    )
    return f(user_indices, item_indices, *utabs, *itabs)


def kernel(user_indices, item_indices, edge_index, weights, user_emb, item_emb):
    src = edge_index[0]
    dst = edge_index[1]
    u1, i1 = _layer(user_emb, item_emb, src, dst, weights)
    u2, i2 = _layer(u1, i1, src, dst, weights)
    u3, i3 = _layer(u2, i2, src, dst, weights)
    return _finalize(user_indices, item_indices,
                     (user_emb, u1, u2, u3), (item_emb, i1, i2, i3))


# speedup vs baseline: 6.2537x; 2.2416x over previous
import jax
import jax.numpy as jnp
from jax import lax
from jax.experimental import pallas as pl
from jax.experimental.pallas import tpu as pltpu
from jax.experimental.pallas import tpu_sc as plsc

N_NODES = 10000
N_PAD = 10240
D = 128
E = 320000
B = 4096

NC = 2
NS = 16
L = 16

E_PER_TILE = E // NS
CHUNK = 40
NB = 5
EPG = NB * CHUNK
NGROUPS = E_PER_TILE // EPG
ROWS_PER_TILE = N_PAD // NS
WB = 64

_mesh = plsc.VectorSubcoreMesh(core_axis_name="c", subcore_axis_name="s")


def _layer_body(u_emb, i_emb, src, dst, w_hbm, u_out, i_out,
                acc, gbuf0, gbuf1, wbuf0, wbuf1,
                sib00, sib01, sib02, sib03, sib04,
                sib10, sib11, sib12, sib13, sib14,
                rows0, rows1, rows2, rows3, rows4,
                zb, i_sem, g_sem, s_sem):
    gbufs = (gbuf0, gbuf1)
    wbufs = (wbuf0, wbuf1)
    sibufs = ((sib00, sib01, sib02, sib03, sib04),
              (sib10, sib11, sib12, sib13, sib14))
    rows = (rows0, rows1, rows2, rows3, rows4)
    c = lax.axis_index("c")
    s = lax.axis_index("s")

    def zrow(r, _):
        for j in range(D // L):
            zb[r, pl.ds(j * L, L)] = jnp.zeros((L,), jnp.float32)
        return 0

    lax.fori_loop(0, WB, zrow, 0)
    for k in range(ROWS_PER_TILE // WB):
        pltpu.sync_copy(zb, acc.at[pl.ds(s * ROWS_PER_TILE + k * WB, WB)])
    plsc.subcore_barrier()

    def do_pass(gather_tab, scat_idx, gath_idx, out_hbm):
        base0 = s * E_PER_TILE

        def idx_copies(p, g):
            cb = base0 + g * EPG
            cps = [
                pltpu.make_async_copy(gath_idx.at[pl.ds(cb, EPG)],
                                      gbufs[p], i_sem),
                pltpu.make_async_copy(w_hbm.at[pl.ds(cb, EPG)],
                                      wbufs[p], i_sem),
            ]
            for b in range(NB):
                cps.append(pltpu.make_async_copy(
                    scat_idx.at[pl.ds(cb + b * CHUNK, CHUNK)],
                    sibufs[p][b], i_sem))
            return cps

        def gather_copy(p, b):
            gsl = gbufs[p].at[pl.ds(b * CHUNK, CHUNK)]
            return pltpu.make_async_copy(gather_tab.at[gsl], rows[b],
                                         g_sem.at[b])

        def scatter_wait(p, b):
            pltpu.make_async_copy(rows[b], acc.at[sibufs[p][b]],
                                  s_sem.at[b]).wait()

        def scale(p, b):
            rb = rows[b]

            @pl.loop(0, CHUNK)
            def _(r):
                widx = jnp.full((L,), b * CHUNK, dtype=jnp.int32) + r
                wv = plsc.load_gather(wbufs[p], [widx])
                for j in range(D // L):
                    sl = pl.ds(j * L, L)
                    rb[r, sl] = rb[r, sl] * wv

        def group(p, g, first, last):
            for cp in idx_copies(p, g):
                cp.wait()
            for b in range(NB):
                if not first:
                    scatter_wait(1 - p, b)
                gather_copy(p, b).start()
            if not last:
                for cp in idx_copies(1 - p, g + 1):
                    cp.start()
            for b in range(NB):
                gather_copy(p, b).wait()
                scale(p, b)
                pltpu.async_copy(rows[b], acc.at[sibufs[p][b]],
                                 s_sem.at[b], add=True)
            if last:
                for b in range(NB):
                    scatter_wait(p, b)

        for cp in idx_copies(0, 0):
            cp.start()
        group(0, 0, first=True, last=False)

        @pl.loop(0, (NGROUPS - 2) // 2)
        def _(h):
            group(1, 2 * h + 1, first=False, last=False)
            group(0, 2 * h + 2, first=False, last=False)

        group(1, NGROUPS - 1, first=False, last=True)

        plsc.subcore_barrier()
        for k in range(ROWS_PER_TILE // WB):
            row0 = s * ROWS_PER_TILE + k * WB
            pltpu.sync_copy(acc.at[pl.ds(row0, WB)], zb)
            pltpu.sync_copy(zb, out_hbm.at[pl.ds(row0, WB)])

    @pl.when(c == 0)
    def _():
        do_pass(i_emb, src, dst, u_out)

    @pl.when(c == 1)
    def _():
        do_pass(u_emb, dst, src, i_out)


@jax.jit
def _layer(u_emb, i_emb, src, dst, weights):
    f = pl.kernel(
        _layer_body,
        compiler_params=pltpu.CompilerParams(needs_layout_passes=False),
        out_type=(
            jax.ShapeDtypeStruct((N_PAD, D), jnp.float32),
            jax.ShapeDtypeStruct((N_PAD, D), jnp.float32),
        ),
        mesh=_mesh,
        scratch_types=[
            pltpu.VMEM_SHARED((N_PAD, D), jnp.float32),
            pltpu.VMEM((EPG,), jnp.int32),
            pltpu.VMEM((EPG,), jnp.int32),
            pltpu.VMEM((EPG,), jnp.float32),
            pltpu.VMEM((EPG,), jnp.float32),
            *([pltpu.VMEM((CHUNK,), jnp.int32)] * (2 * NB)),
            *([pltpu.VMEM((CHUNK, D), jnp.float32)] * NB),
            pltpu.VMEM((WB, D), jnp.float32),
            pltpu.SemaphoreType.DMA,
            pltpu.SemaphoreType.DMA((NB,)),
            pltpu.SemaphoreType.DMA((NB,)),
        ],
    )
    return f(u_emb, i_emb, src, dst, weights)


P_PER_TILE = B // (NC * NS)
PC = 64


def _final_body(uidx_hbm, iidx_hbm, u0, u1, u2, u3, i0, i1, i2, i3, out_hbm,
                uidx_v, iidx_v, ubuf, ibuf, out_v, sem):
    c = lax.axis_index("c")
    s = lax.axis_index("s")
    wid = s * NC + c
    base = wid * P_PER_TILE

    utabs = (u0, u1, u2, u3)
    itabs = (i0, i1, i2, i3)
    for co in range(P_PER_TILE // PC):
        pltpu.sync_copy(uidx_hbm.at[pl.ds(base + co * PC, PC)], uidx_v)
        pltpu.sync_copy(iidx_hbm.at[pl.ds(base + co * PC, PC)], iidx_v)
        cps = []
        for t in range(4):
            cps.append(pltpu.make_async_copy(utabs[t].at[uidx_v],
                                             ubuf.at[t], sem))
            cps.append(pltpu.make_async_copy(itabs[t].at[iidx_v],
                                             ibuf.at[t], sem))
        for cp in cps:
            cp.start()
        for cp in cps:
            cp.wait()
        for g in range(PC // L):
            rows_iota = lax.iota(jnp.int32, L) + g * L

            def dstep(d, acc):
                col = jnp.full((L,), d, dtype=jnp.int32)
                us = plsc.load_gather(
                    ubuf, [jnp.zeros((L,), jnp.int32), rows_iota, col])
                is_ = plsc.load_gather(
                    ibuf, [jnp.zeros((L,), jnp.int32), rows_iota, col])
                for t in range(1, 4):
                    tt = jnp.full((L,), t, dtype=jnp.int32)
                    us = us + plsc.load_gather(ubuf, [tt, rows_iota, col])
                    is_ = is_ + plsc.load_gather(ibuf, [tt, rows_iota, col])
                return acc + us * is_

            accv = lax.fori_loop(0, D, dstep, jnp.zeros((L,), jnp.float32),
                                 unroll=2)
            out_v[pl.ds(co * PC + g * L, L)] = accv * (1.0 / 16.0)
    pltpu.sync_copy(out_v, out_hbm.at[pl.ds(base, P_PER_TILE)])


@jax.jit
def _finalize(user_indices, item_indices, utabs, itabs):
    f = pl.kernel(
        _final_body,
        compiler_params=pltpu.CompilerParams(needs_layout_passes=False),
        out_type=jax.ShapeDtypeStruct((B,), jnp.float32),
        mesh=_mesh,
        scratch_types=[
            pltpu.VMEM((PC,), jnp.int32),
            pltpu.VMEM((PC,), jnp.int32),
            pltpu.VMEM((4, PC, D), jnp.float32),
            pltpu.VMEM((4, PC, D), jnp.float32),
            pltpu.VMEM((P_PER_TILE,), jnp.float32),
            pltpu.SemaphoreType.DMA,
        ],
    )
    return f(user_indices, item_indices, *utabs, *itabs)


def kernel(user_indices, item_indices, edge_index, weights, user_emb, item_emb):
    src = edge_index[0]
    dst = edge_index[1]
    u1, i1 = _layer(user_emb, item_emb, src, dst, weights)
    u2, i2 = _layer(u1, i1, src, dst, weights)
    u3, i3 = _layer(u2, i2, src, dst, weights)
    return _finalize(user_indices, item_indices,
                     (user_emb, u1, u2, u3), (item_emb, i1, i2, i3))
